# Initial kernel scaffold; baseline (speedup 1.0000x reference)
import functools

import jax
import jax.numpy as jnp
from jax import lax
from jax.experimental import pallas as pl
from jax.experimental.pallas import tpu as pltpu
from jax.experimental.pallas import tpu_sc as plsc

NW = 32
CHUNK = 100


def _pos_encoding(length, d_model):
    depth = d_model / 2
    pos = jnp.arange(0, length, dtype=jnp.float32)[:, None]
    i = jnp.arange(0, depth, dtype=jnp.float32)
    angle = pos / jnp.power(10000.0, 2.0 * i / depth)
    return jnp.concatenate([jnp.sin(angle), jnp.cos(angle)], axis=-1)


def _scale_body(w_ref, o_ref):
    o_ref[...] = w_ref[...] * 8.0


def _prescale(W):
    V, D = W.shape
    blk = 2000
    return pl.pallas_call(
        _scale_body,
        grid=(V // blk,),
        in_specs=[pl.BlockSpec((blk, D), lambda i: (i, 0))],
        out_specs=pl.BlockSpec((blk, D), lambda i: (i, 0)),
        out_shape=jax.ShapeDtypeStruct((V, D), jnp.float32),
    )(W)


def kernel(x, W):
    B, L = x.shape
    V, D = W.shape
    W8 = _prescale(W)
    pe = _pos_encoding(L, D).reshape(L // CHUNK, CHUNK, D)
    nchunks = B * L // (NW * CHUNK)
    xr = x.reshape(NW, nchunks, CHUNK)
    nper = L // CHUNK

    mesh = plsc.VectorSubcoreMesh(core_axis_name="c", subcore_axis_name="s")

    @functools.partial(
        pl.kernel,
        out_type=jax.ShapeDtypeStruct((NW, nchunks, CHUNK, D), jnp.float32),
        mesh=mesh,
        scratch_types=[
            pltpu.VMEM((nchunks, CHUNK), jnp.int32),
            pltpu.VMEM((CHUNK, D), jnp.float32),
            pltpu.SemaphoreType.DMA,
        ],
    )
    def sc_run(w_hbm, x_hbm, pe_hbm, out_hbm, idx_v, buf, sem):
        wid = lax.axis_index("s") * 2 + lax.axis_index("c")
        pltpu.sync_copy(x_hbm.at[wid], idx_v)

        def body(j, carry):
            pltpu.sync_copy(pe_hbm.at[lax.rem(j, nper)], buf)
            pltpu.async_copy(w_hbm.at[idx_v.at[j]], buf, sem, add=True).wait()
            pltpu.sync_copy(buf, out_hbm.at[wid, j])
            return carry

        lax.fori_loop(0, nchunks, body, 0)

    out = sc_run(W8, xr, pe)
    return out.reshape(B, L, D)


# speedup vs baseline: 2.2764x; 2.2764x over previous
import functools

import jax
import jax.numpy as jnp
from jax import lax
from jax.experimental import pallas as pl
from jax.experimental.pallas import tpu as pltpu
from jax.experimental.pallas import tpu_sc as plsc

NW = 32
CHUNK = 100


def _pos_encoding(length, d_model):
    depth = d_model / 2
    pos = jnp.arange(0, length, dtype=jnp.float32)[:, None]
    i = jnp.arange(0, depth, dtype=jnp.float32)
    angle = pos / jnp.power(10000.0, 2.0 * i / depth)
    return jnp.concatenate([jnp.sin(angle), jnp.cos(angle)], axis=-1)


def _scale_body(w_ref, o_ref):
    o_ref[...] = w_ref[...] * 8.0


def _prescale(W):
    V, D = W.shape
    blk = 2000
    return pl.pallas_call(
        _scale_body,
        grid=(V // blk,),
        in_specs=[pl.BlockSpec((blk, D), lambda i: (i, 0))],
        out_specs=pl.BlockSpec((blk, D), lambda i: (i, 0)),
        out_shape=jax.ShapeDtypeStruct((V, D), jnp.float32),
    )(W)


def kernel(x, W):
    B, L = x.shape
    V, D = W.shape
    W8 = _prescale(W)
    pe = _pos_encoding(L, D).reshape(L // CHUNK, CHUNK, D)
    nchunks = B * L // (NW * CHUNK)
    xr = x.reshape(NW, nchunks, CHUNK)
    nper = L // CHUNK

    mesh = plsc.VectorSubcoreMesh(core_axis_name="c", subcore_axis_name="s")

    @functools.partial(
        pl.kernel,
        out_type=jax.ShapeDtypeStruct((NW, nchunks, CHUNK, D), jnp.float32),
        mesh=mesh,
        scratch_types=[
            pltpu.VMEM((nchunks, CHUNK), jnp.int32),
            pltpu.VMEM((CHUNK, D), jnp.float32),
            pltpu.SemaphoreType.DMA,
        ],
        compiler_params=pltpu.CompilerParams(use_tc_tiling_on_sc=False),
    )
    def sc_run(w_hbm, x_hbm, pe_hbm, out_hbm, idx_v, buf, sem):
        wid = lax.axis_index("s") * 2 + lax.axis_index("c")
        pltpu.sync_copy(x_hbm.at[wid], idx_v)

        def body(j, carry):
            pltpu.sync_copy(pe_hbm.at[lax.rem(j, nper)], buf)
            pltpu.async_copy(w_hbm.at[idx_v.at[j]], buf, sem, add=True).wait()
            pltpu.sync_copy(buf, out_hbm.at[wid, j])
            return carry

        lax.fori_loop(0, nchunks, body, 0)

    out = sc_run(W8, xr, pe)
    return out.reshape(B, L, D)


# speedup vs baseline: 2.3646x; 1.0388x over previous
import functools

import jax
import jax.numpy as jnp
from jax import lax
from jax.experimental import pallas as pl
from jax.experimental.pallas import tpu as pltpu
from jax.experimental.pallas import tpu_sc as plsc

NW = 32
CHUNK = 100


def _pos_encoding(length, d_model):
    depth = d_model / 2
    pos = jnp.arange(0, length, dtype=jnp.float32)[:, None]
    i = jnp.arange(0, depth, dtype=jnp.float32)
    angle = pos / jnp.power(10000.0, 2.0 * i / depth)
    return jnp.concatenate([jnp.sin(angle), jnp.cos(angle)], axis=-1)


def _scale_body(w_ref, o_ref):
    o_ref[...] = w_ref[...] * 8.0


def _prescale(W):
    V, D = W.shape
    blk = 2000
    return pl.pallas_call(
        _scale_body,
        grid=(V // blk,),
        in_specs=[pl.BlockSpec((blk, D), lambda i: (i, 0))],
        out_specs=pl.BlockSpec((blk, D), lambda i: (i, 0)),
        out_shape=jax.ShapeDtypeStruct((V, D), jnp.float32),
    )(W)


def kernel(x, W):
    B, L = x.shape
    V, D = W.shape
    W8 = _prescale(W)
    pe = _pos_encoding(L, D).reshape(L // CHUNK, CHUNK, D)
    nchunks = B * L // (NW * CHUNK)
    xr = x.reshape(NW, nchunks, CHUNK)
    nper = L // CHUNK

    mesh = plsc.VectorSubcoreMesh(core_axis_name="c", subcore_axis_name="s")
    NBUF = 4

    @functools.partial(
        pl.kernel,
        out_type=jax.ShapeDtypeStruct((NW, nchunks, CHUNK, D), jnp.float32),
        mesh=mesh,
        scratch_types=[
            pltpu.VMEM((nchunks, CHUNK), jnp.int32),
            pltpu.VMEM((NBUF, CHUNK, D), jnp.float32),
            pltpu.SemaphoreType.DMA((NBUF,)),
            pltpu.SemaphoreType.DMA((NBUF,)),
            pltpu.SemaphoreType.DMA((NBUF,)),
        ],
        compiler_params=pltpu.CompilerParams(use_tc_tiling_on_sc=False),
    )
    def sc_run(w_hbm, x_hbm, pe_hbm, out_hbm, idx_v, buf, isem, gsem, osem):
        wid = lax.axis_index("s") * 2 + lax.axis_index("c")
        pltpu.sync_copy(x_hbm.at[wid], idx_v)

        def init_start(c):
            s = lax.rem(c, NBUF)
            pltpu.async_copy(pe_hbm.at[lax.rem(c, nper)], buf.at[s], isem.at[s])

        def init_wait(c):
            s = lax.rem(c, NBUF)
            pltpu.make_async_copy(pe_hbm.at[lax.rem(c, nper)], buf.at[s],
                                  isem.at[s]).wait()

        def gather_start(c):
            s = lax.rem(c, NBUF)
            pltpu.async_copy(w_hbm.at[idx_v.at[c]], buf.at[s], gsem.at[s],
                             add=True)

        def gather_wait(c):
            s = lax.rem(c, NBUF)
            pltpu.make_async_copy(pe_hbm.at[0], buf.at[s], gsem.at[s]).wait()

        def out_start(c):
            s = lax.rem(c, NBUF)
            pltpu.async_copy(buf.at[s], out_hbm.at[wid, c], osem.at[s])

        def out_wait(c):
            s = lax.rem(c, NBUF)
            pltpu.make_async_copy(buf.at[s], out_hbm.at[wid, c],
                                  osem.at[s]).wait()

        init_start(0)
        init_start(1)

        def body(j, carry):
            @pl.when(jnp.logical_and(j >= 1, j <= nchunks - 1 + 1))
            def _():
                gather_wait(j - 1)
                out_start(j - 1)

            @pl.when(jnp.logical_and(j >= 2, j <= nchunks + 1))
            def _():
                out_wait(j - 2)

            @pl.when(j <= nchunks - 3)
            def _():
                init_start(j + 2)

            @pl.when(j <= nchunks - 1)
            def _():
                init_wait(j)
                gather_start(j)

            return carry

        lax.fori_loop(0, nchunks + 2, body, 0)

    out = sc_run(W8, xr, pe)
    return out.reshape(B, L, D)


# speedup vs baseline: 2.3696x; 1.0021x over previous
import functools

import jax
import jax.numpy as jnp
from jax import lax
from jax.experimental import pallas as pl
from jax.experimental.pallas import tpu as pltpu
from jax.experimental.pallas import tpu_sc as plsc

NW = 32
CHUNK = 100


def _pos_encoding(length, d_model):
    depth = d_model / 2
    pos = jnp.arange(0, length, dtype=jnp.float32)[:, None]
    i = jnp.arange(0, depth, dtype=jnp.float32)
    angle = pos / jnp.power(10000.0, 2.0 * i / depth)
    return jnp.concatenate([jnp.sin(angle), jnp.cos(angle)], axis=-1)


def _scale_body(w_ref, o_ref):
    o_ref[...] = w_ref[...] * 8.0


def _prescale(W):
    V, D = W.shape
    blk = 2000
    return pl.pallas_call(
        _scale_body,
        grid=(V // blk,),
        in_specs=[pl.BlockSpec((blk, D), lambda i: (i, 0))],
        out_specs=pl.BlockSpec((blk, D), lambda i: (i, 0)),
        out_shape=jax.ShapeDtypeStruct((V, D), jnp.float32),
    )(W)


def kernel(x, W):
    B, L = x.shape
    V, D = W.shape
    W8 = _prescale(W)
    pe = _pos_encoding(L, D).reshape(L // CHUNK, CHUNK, D)
    nchunks = B * L // (NW * CHUNK)
    xr = x.reshape(NW, nchunks, CHUNK)
    nper = L // CHUNK

    mesh = plsc.VectorSubcoreMesh(core_axis_name="c", subcore_axis_name="s")
    NBUF = 4

    @functools.partial(
        pl.kernel,
        out_type=jax.ShapeDtypeStruct((NW, nchunks, CHUNK, D), jnp.float32),
        mesh=mesh,
        scratch_types=[
            pltpu.VMEM((nchunks, CHUNK), jnp.int32),
            pltpu.VMEM((NBUF, CHUNK, D), jnp.float32),
            pltpu.SemaphoreType.DMA((NBUF,)),
            pltpu.SemaphoreType.DMA((NBUF,)),
            pltpu.SemaphoreType.DMA((NBUF,)),
        ],
        compiler_params=pltpu.CompilerParams(use_tc_tiling_on_sc=False),
    )
    def sc_run(w_hbm, x_hbm, pe_hbm, out_hbm, idx_v, buf, isem, gsem, osem):
        wid = lax.axis_index("s") * 2 + lax.axis_index("c")
        pltpu.sync_copy(x_hbm.at[wid], idx_v)

        def init_start(c):
            s = lax.rem(c, NBUF)
            pltpu.async_copy(pe_hbm.at[lax.rem(c, nper)], buf.at[s], isem.at[s])

        def init_wait(c):
            s = lax.rem(c, NBUF)
            pltpu.make_async_copy(pe_hbm.at[lax.rem(c, nper)], buf.at[s],
                                  isem.at[s]).wait()

        def gather_start(c):
            s = lax.rem(c, NBUF)
            pltpu.async_copy(w_hbm.at[idx_v.at[c]], buf.at[s], gsem.at[s],
                             add=True)

        def gather_wait(c):
            s = lax.rem(c, NBUF)
            pltpu.make_async_copy(pe_hbm.at[0], buf.at[s], gsem.at[s]).wait()

        def out_start(c):
            s = lax.rem(c, NBUF)
            pltpu.async_copy(buf.at[s], out_hbm.at[wid, c], osem.at[s])

        def out_wait(c):
            s = lax.rem(c, NBUF)
            pltpu.make_async_copy(buf.at[s], out_hbm.at[wid, c],
                                  osem.at[s]).wait()

        init_start(0)

        def body(j, carry):
            @pl.when(jnp.logical_and(j >= 3, j <= nchunks + 2))
            def _():
                out_wait(j - 3)

            @pl.when(j <= nchunks - 2)
            def _():
                init_start(j + 1)

            @pl.when(j <= nchunks - 1)
            def _():
                init_wait(j)
                gather_start(j)

            @pl.when(jnp.logical_and(j >= 2, j <= nchunks + 1))
            def _():
                gather_wait(j - 2)
                out_start(j - 2)

            return carry

        lax.fori_loop(0, nchunks + 3, body, 0)

    out = sc_run(W8, xr, pe)
    return out.reshape(B, L, D)


# speedup vs baseline: 2.3844x; 1.0062x over previous
import functools

import jax
import jax.numpy as jnp
from jax import lax
from jax.experimental import pallas as pl
from jax.experimental.pallas import tpu as pltpu
from jax.experimental.pallas import tpu_sc as plsc

NW = 32
CHUNK = 100


def _pos_encoding(length, d_model):
    depth = d_model / 2
    pos = jnp.arange(0, length, dtype=jnp.float32)[:, None]
    i = jnp.arange(0, depth, dtype=jnp.float32)
    angle = pos / jnp.power(10000.0, 2.0 * i / depth)
    return jnp.concatenate([jnp.sin(angle), jnp.cos(angle)], axis=-1)


def _scale_body(w_ref, o_ref):
    o_ref[...] = w_ref[...] * 8.0


def _prescale(W):
    V, D = W.shape
    blk = 2000
    return pl.pallas_call(
        _scale_body,
        grid=(V // blk,),
        in_specs=[pl.BlockSpec((blk, D), lambda i: (i, 0))],
        out_specs=pl.BlockSpec((blk, D), lambda i: (i, 0)),
        out_shape=jax.ShapeDtypeStruct((V, D), jnp.float32),
    )(W)


def kernel(x, W):
    B, L = x.shape
    V, D = W.shape
    W8 = _prescale(W)
    pe = _pos_encoding(L, D)
    nseq = B // NW
    nper = L // CHUNK

    mesh = plsc.VectorSubcoreMesh(core_axis_name="c", subcore_axis_name="s")
    NBUF = 4

    @functools.partial(
        pl.kernel,
        out_type=jax.ShapeDtypeStruct((B, L, D), jnp.float32),
        mesh=mesh,
        scratch_types=[
            pltpu.VMEM((nseq, L), jnp.int32),
            pltpu.VMEM((NBUF, L, D), jnp.float32),
            pltpu.SemaphoreType.DMA((NBUF,)),
            pltpu.SemaphoreType.DMA((NBUF,)),
            pltpu.SemaphoreType.DMA((NBUF,)),
        ],
        compiler_params=pltpu.CompilerParams(use_tc_tiling_on_sc=False),
    )
    def sc_run(w_hbm, x_hbm, pe_hbm, out_hbm, idx_v, buf, isem, gsem, osem):
        wid = lax.axis_index("s") * 2 + lax.axis_index("c")
        pltpu.sync_copy(x_hbm.at[pl.ds(wid * nseq, nseq)], idx_v)

        def init_start(c):
            s = lax.rem(c, NBUF)
            pltpu.async_copy(pe_hbm, buf.at[s], isem.at[s])

        def init_wait(c):
            s = lax.rem(c, NBUF)
            pltpu.make_async_copy(pe_hbm, buf.at[s], isem.at[s]).wait()

        def gather_start(c):
            s = lax.rem(c, NBUF)
            pltpu.async_copy(w_hbm.at[idx_v.at[c]], buf.at[s], gsem.at[s],
                             add=True)

        def gather_wait(c):
            s = lax.rem(c, NBUF)
            pltpu.make_async_copy(pe_hbm, buf.at[s], gsem.at[s]).wait()

        def out_start(c):
            s = lax.rem(c, NBUF)
            pltpu.async_copy(buf.at[s], out_hbm.at[wid * nseq + c], osem.at[s])

        def out_wait(c):
            s = lax.rem(c, NBUF)
            pltpu.make_async_copy(buf.at[s], out_hbm.at[wid * nseq + c],
                                  osem.at[s]).wait()

        init_start(0)

        def body(j, carry):
            @pl.when(jnp.logical_and(j >= 3, j <= nseq + 2))
            def _():
                out_wait(j - 3)

            @pl.when(j <= nseq - 2)
            def _():
                init_start(j + 1)

            @pl.when(j <= nseq - 1)
            def _():
                init_wait(j)
                gather_start(j)

            @pl.when(jnp.logical_and(j >= 2, j <= nseq + 1))
            def _():
                gather_wait(j - 2)
                out_start(j - 2)

            return carry

        lax.fori_loop(0, nseq + 3, body, 0)

    return sc_run(W8, x, pe)
